# baseline (speedup 1.0000x reference)
import functools

import jax
import jax.numpy as jnp
from jax import lax
from jax.experimental import pallas as pl
from jax.experimental.pallas import tpu as pltpu
from jax.experimental.pallas import tpu_sc as plsc

B = 16384
D = 32
N_ROWS = 1000000
PACK = 4
W = D * PACK
PROWS = N_ROWS // PACK
NC = 2
NS = 16
NW = NC * NS
BPW = B // NW
CHUNK = 128
NCH = BPW // CHUNK
NBUF = 3


RL = 512
RGRID = (N_ROWS + PACK * RL - 1) // (PACK * RL)


def _repack_tc(t0_ref, t1_ref, t2_ref, t3_ref, out_ref):
    out_ref[...] = jnp.concatenate(
        [jnp.transpose(t0_ref[...]), jnp.transpose(t1_ref[...]),
         jnp.transpose(t2_ref[...]), jnp.transpose(t3_ref[...])], axis=1)


def _repack(table_t):
    LASTB = (N_ROWS - 1) // RL
    qmaps = [
        (lambda k: (lambda i: (0, jnp.minimum(PACK * i + k, LASTB))))(k)
        for k in range(PACK)
    ]
    return pl.pallas_call(
        _repack_tc,
        grid=(RGRID,),
        in_specs=[pl.BlockSpec((D, RL), m) for m in qmaps],
        out_specs=pl.BlockSpec((RL, W), lambda i: (i, 0)),
        out_shape=jax.ShapeDtypeStruct((RGRID * RL, W), jnp.float32),
    )(table_t, table_t, table_t, table_t)


_sc_mesh = plsc.VectorSubcoreMesh(core_axis_name="c", subcore_axis_name="s")

_out_row = jax.ShapeDtypeStruct((B, W), jnp.float32)
PROWS_PAD = ((N_ROWS + PACK * RL - 1) // (PACK * RL)) * RL


@functools.partial(
    pl.kernel,
    mesh=_sc_mesh,
    out_type=(_out_row, _out_row, _out_row, _out_row),
    scratch_types=(
        pltpu.VMEM((NCH, CHUNK), jnp.int32),
        pltpu.VMEM((NCH, CHUNK), jnp.int32),
        pltpu.VMEM((NBUF, CHUNK, W), jnp.float32),
        pltpu.SemaphoreType.DMA((NBUF,)),
        pltpu.SemaphoreType.DMA((NBUF,)),
    ),
)
def _gather_sc(uidx_hbm, iidx_hbm, mfu_hbm, mfi_hbm, mlu_hbm, mli_hbm,
               out_mfu, out_mfi, out_mlu, out_mli,
               uidx_v, iidx_v, bufs, sem_in, sem_out):
    wid = lax.axis_index("s") * NC + lax.axis_index("c")
    row0 = wid * NCH
    base = wid * BPW
    pltpu.sync_copy(uidx_hbm.at[pl.ds(row0, NCH)], uidx_v)
    pltpu.sync_copy(iidx_hbm.at[pl.ds(row0, NCH)], iidx_v)

    plan = []
    for tbl, out, idx in (
        (mfu_hbm, out_mfu, uidx_v),
        (mfi_hbm, out_mfi, iidx_v),
        (mlu_hbm, out_mlu, uidx_v),
        (mli_hbm, out_mli, iidx_v),
    ):
        for c in range(NCH):
            plan.append((tbl, out, idx, c))

    n = len(plan)
    in_descs = [None] * n
    out_descs = [None] * n

    def fire_in(r):
        tbl, _, idx, c = plan[r]
        in_descs[r] = pltpu.async_copy(tbl.at[idx.at[c]], bufs.at[r % NBUF],
                                       sem_in.at[r % NBUF])

    fire_in(0)
    for r in range(n):
        if r + 1 < n:
            if r + 1 >= NBUF:
                out_descs[r + 1 - NBUF].wait()
            fire_in(r + 1)
        in_descs[r].wait()
        _, out, _, c = plan[r]
        out_descs[r] = pltpu.async_copy(
            bufs.at[r % NBUF], out.at[pl.ds(base + c * CHUNK, CHUNK)],
            sem_out.at[r % NBUF])
    for r in range(n - NBUF + 1, n):
        out_descs[r].wait()


BB = 2048


def _extract(buf, sel):
    acc = jnp.where(sel == 0, buf[:, 0:D], 0.0)
    for k in range(1, PACK):
        acc = acc + jnp.where(sel == k, buf[:, k * D:(k + 1) * D], 0.0)
    return acc


def _dense_tc(selu_ref, seli_ref, mfu_ref, mfi_ref, mlu_ref, mli_ref,
              w1u_ref, w1i_ref, b1_ref, w2t_ref, b2_ref,
              wpm_ref, wph_ref, bp_ref, out_ref):
    su = selu_ref[...]
    si = seli_ref[...]
    dot = functools.partial(jnp.dot, preferred_element_type=jnp.float32,
                            precision=lax.Precision.HIGHEST)
    mfu = _extract(mfu_ref[...], su)
    mfi = _extract(mfi_ref[...], si)
    mlu = _extract(mlu_ref[...], su)
    mli = _extract(mli_ref[...], si)
    h1 = dot(mlu, w1u_ref[...]) + dot(mli, w1i_ref[...])
    h1 = jnp.maximum(h1 + b1_ref[...], 0.0)
    h2 = jnp.maximum(dot(h1, w2t_ref[...]) + b2_ref[...], 0.0)
    mf = mfu * mfi
    acc = jnp.sum(mf * wpm_ref[...], axis=1) + jnp.sum(h2 * wph_ref[...], axis=1)
    out_ref[...] = acc + bp_ref[0, 0]


def kernel(user_idx, item_idx, mf_user_w, mf_item_w, mlp_user_w, mlp_item_w,
           W1, b1, W2, b2, Wp, bp):
    ui = user_idx.astype(jnp.int32)
    ii = item_idx.astype(jnp.int32)
    uidx4 = (((ui >> 11) << 9) + (ui & 511)).reshape(B // CHUNK, CHUNK)
    iidx4 = (((ii >> 11) << 9) + (ii & 511)).reshape(B // CHUNK, CHUNK)
    selu = ((ui >> 9) & 3).reshape(B, 1)
    seli = ((ii >> 9) & 3).reshape(B, 1)

    mfu_p = _repack(mf_user_w.T)
    mfi_p = _repack(mf_item_w.T)
    mlu_p = _repack(mlp_user_w.T)
    mli_p = _repack(mlp_item_w.T)

    mfu, mfi, mlu, mli = _gather_sc(uidx4, iidx4, mfu_p, mfi_p, mlu_p, mli_p)

    w1u = W1[:, :D].T
    w1i = W1[:, D:].T
    w2t = W2.T
    b1r = b1.reshape(1, -1)
    b2r = b2.reshape(1, -1)
    wpm = Wp[:, :D]
    wph = Wp[:, D:]
    bpr = bp.reshape(1, 1)

    grid = B // BB
    full = lambda i: (0, 0)
    row = lambda i: (i, 0)
    out = pl.pallas_call(
        _dense_tc,
        grid=(grid,),
        in_specs=[
            pl.BlockSpec((BB, 1), row),
            pl.BlockSpec((BB, 1), row),
            pl.BlockSpec((BB, W), row),
            pl.BlockSpec((BB, W), row),
            pl.BlockSpec((BB, W), row),
            pl.BlockSpec((BB, W), row),
            pl.BlockSpec((D, 32), full),
            pl.BlockSpec((D, 32), full),
            pl.BlockSpec((1, 32), full),
            pl.BlockSpec((D, 16), full),
            pl.BlockSpec((1, 16), full),
            pl.BlockSpec((1, D), full),
            pl.BlockSpec((1, 16), full),
            pl.BlockSpec((1, 1), full),
        ],
        out_specs=pl.BlockSpec((BB,), lambda i: (i,)),
        out_shape=jax.ShapeDtypeStruct((B,), jnp.float32),
    )(selu, seli, mfu, mfi, mlu, mli, w1u, w1i, b1r, w2t, b2r, wpm, wph, bpr)
    return out


# speedup vs baseline: 1.6244x; 1.6244x over previous
import functools

import jax
import jax.numpy as jnp
from jax import lax
from jax.experimental import pallas as pl
from jax.experimental.pallas import tpu as pltpu
from jax.experimental.pallas import tpu_sc as plsc

B = 16384
D = 32
N_ROWS = 1000000
PACK = 4
W = D * PACK
PROWS = N_ROWS // PACK
NC = 2
NS = 16
NW = NC * NS
BPW = B // NW
CHUNK = 128
NCH = BPW // CHUNK
NBUF = 3


RL = 8192
RIN = PACK * RL
RGRID = (N_ROWS + RIN - 1) // RIN


def _repack_tc(t_ref, out_ref):
    out_ref[...] = jnp.concatenate(
        [jnp.transpose(t_ref[:, k * RL:(k + 1) * RL]) for k in range(PACK)],
        axis=1)


def _repack(table_t):
    return pl.pallas_call(
        _repack_tc,
        grid=(RGRID,),
        in_specs=[pl.BlockSpec((D, RIN), lambda i: (0, i))],
        out_specs=pl.BlockSpec((RL, W), lambda i: (i, 0)),
        out_shape=jax.ShapeDtypeStruct((RGRID * RL, W), jnp.float32),
    )(table_t)


_sc_mesh = plsc.VectorSubcoreMesh(core_axis_name="c", subcore_axis_name="s")

_out_row = jax.ShapeDtypeStruct((B, W), jnp.float32)
PROWS_PAD = ((N_ROWS + PACK * RL - 1) // (PACK * RL)) * RL


@functools.partial(
    pl.kernel,
    mesh=_sc_mesh,
    out_type=(_out_row, _out_row, _out_row, _out_row),
    scratch_types=(
        pltpu.VMEM((NCH, CHUNK), jnp.int32),
        pltpu.VMEM((NCH, CHUNK), jnp.int32),
        pltpu.VMEM((NBUF, CHUNK, W), jnp.float32),
        pltpu.SemaphoreType.DMA((NBUF,)),
        pltpu.SemaphoreType.DMA((NBUF,)),
    ),
)
def _gather_sc(uidx_hbm, iidx_hbm, mfu_hbm, mfi_hbm, mlu_hbm, mli_hbm,
               out_mfu, out_mfi, out_mlu, out_mli,
               uidx_v, iidx_v, bufs, sem_in, sem_out):
    wid = lax.axis_index("s") * NC + lax.axis_index("c")
    row0 = wid * NCH
    base = wid * BPW
    pltpu.sync_copy(uidx_hbm.at[pl.ds(row0, NCH)], uidx_v)
    pltpu.sync_copy(iidx_hbm.at[pl.ds(row0, NCH)], iidx_v)

    plan = []
    for tbl, out, idx in (
        (mfu_hbm, out_mfu, uidx_v),
        (mfi_hbm, out_mfi, iidx_v),
        (mlu_hbm, out_mlu, uidx_v),
        (mli_hbm, out_mli, iidx_v),
    ):
        for c in range(NCH):
            plan.append((tbl, out, idx, c))

    n = len(plan)
    in_descs = [None] * n
    out_descs = [None] * n

    def fire_in(r):
        tbl, _, idx, c = plan[r]
        in_descs[r] = pltpu.async_copy(tbl.at[idx.at[c]], bufs.at[r % NBUF],
                                       sem_in.at[r % NBUF])

    fire_in(0)
    for r in range(n):
        if r + 1 < n:
            if r + 1 >= NBUF:
                out_descs[r + 1 - NBUF].wait()
            fire_in(r + 1)
        in_descs[r].wait()
        _, out, _, c = plan[r]
        out_descs[r] = pltpu.async_copy(
            bufs.at[r % NBUF], out.at[pl.ds(base + c * CHUNK, CHUNK)],
            sem_out.at[r % NBUF])
    for r in range(n - NBUF + 1, n):
        out_descs[r].wait()


BB = 2048


def _extract(buf, sel):
    acc = jnp.where(sel == 0, buf[:, 0:D], 0.0)
    for k in range(1, PACK):
        acc = acc + jnp.where(sel == k, buf[:, k * D:(k + 1) * D], 0.0)
    return acc


def _dense_tc(selu_ref, seli_ref, mfu_ref, mfi_ref, mlu_ref, mli_ref,
              w1u_ref, w1i_ref, b1_ref, w2t_ref, b2_ref,
              wpm_ref, wph_ref, bp_ref, out_ref):
    su = selu_ref[...]
    si = seli_ref[...]
    dot = functools.partial(jnp.dot, preferred_element_type=jnp.float32,
                            precision=lax.Precision.HIGHEST)
    mfu = _extract(mfu_ref[...], su)
    mfi = _extract(mfi_ref[...], si)
    mlu = _extract(mlu_ref[...], su)
    mli = _extract(mli_ref[...], si)
    h1 = dot(mlu, w1u_ref[...]) + dot(mli, w1i_ref[...])
    h1 = jnp.maximum(h1 + b1_ref[...], 0.0)
    h2 = jnp.maximum(dot(h1, w2t_ref[...]) + b2_ref[...], 0.0)
    mf = mfu * mfi
    acc = jnp.sum(mf * wpm_ref[...], axis=1) + jnp.sum(h2 * wph_ref[...], axis=1)
    out_ref[...] = acc + bp_ref[0, 0]


def kernel(user_idx, item_idx, mf_user_w, mf_item_w, mlp_user_w, mlp_item_w,
           W1, b1, W2, b2, Wp, bp):
    ui = user_idx.astype(jnp.int32)
    ii = item_idx.astype(jnp.int32)
    uidx4 = (((ui >> 15) << 13) + (ui & (RL - 1))).reshape(B // CHUNK, CHUNK)
    iidx4 = (((ii >> 15) << 13) + (ii & (RL - 1))).reshape(B // CHUNK, CHUNK)
    selu = ((ui >> 13) & 3).reshape(B, 1)
    seli = ((ii >> 13) & 3).reshape(B, 1)

    mfu_p = _repack(mf_user_w.T)
    mfi_p = _repack(mf_item_w.T)
    mlu_p = _repack(mlp_user_w.T)
    mli_p = _repack(mlp_item_w.T)

    mfu, mfi, mlu, mli = _gather_sc(uidx4, iidx4, mfu_p, mfi_p, mlu_p, mli_p)

    w1u = W1[:, :D].T
    w1i = W1[:, D:].T
    w2t = W2.T
    b1r = b1.reshape(1, -1)
    b2r = b2.reshape(1, -1)
    wpm = Wp[:, :D]
    wph = Wp[:, D:]
    bpr = bp.reshape(1, 1)

    grid = B // BB
    full = lambda i: (0, 0)
    row = lambda i: (i, 0)
    out = pl.pallas_call(
        _dense_tc,
        grid=(grid,),
        in_specs=[
            pl.BlockSpec((BB, 1), row),
            pl.BlockSpec((BB, 1), row),
            pl.BlockSpec((BB, W), row),
            pl.BlockSpec((BB, W), row),
            pl.BlockSpec((BB, W), row),
            pl.BlockSpec((BB, W), row),
            pl.BlockSpec((D, 32), full),
            pl.BlockSpec((D, 32), full),
            pl.BlockSpec((1, 32), full),
            pl.BlockSpec((D, 16), full),
            pl.BlockSpec((1, 16), full),
            pl.BlockSpec((1, D), full),
            pl.BlockSpec((1, 16), full),
            pl.BlockSpec((1, 1), full),
        ],
        out_specs=pl.BlockSpec((BB,), lambda i: (i,)),
        out_shape=jax.ShapeDtypeStruct((B,), jnp.float32),
    )(selu, seli, mfu, mfi, mlu, mli, w1u, w1i, b1r, w2t, b2r, wpm, wph, bpr)
    return out
